# baseline (speedup 1.0000x reference)
import numpy as np

import jax
import jax.numpy as jnp
from jax import lax
from jax.experimental import pallas as pl
from jax.experimental.pallas import tpu as pltpu
from jax.experimental.pallas import tpu_sc as plsc

H = 512
W = 512
B = 4
NF = 100000
NTAB = B * NF
HWPIX = H * W
NP = B * HWPIX

NC, NS, L = 2, 16, 16
NW = NC * NS
PIX_PER_W = NP // NW
C = 512
NG = C // L
IDXB = 128
NIDX = C // IDXB
NCHUNK = PIX_PER_W // C

_MUL = np.uint32(2654435761)


def _hash_pix(pvec_u32):
    hsh = pvec_u32 * _MUL
    tri = (hsh % np.uint32(NTAB)).astype(jnp.int32)
    valid = (hsh % np.uint32(7)) != np.uint32(0)
    validf = jnp.where(valid, np.float32(1.0), np.float32(0.0))
    b0 = ((hsh >> 3) % np.uint32(1024)).astype(jnp.float32) + 1.0
    b1 = ((hsh >> 13) % np.uint32(1024)).astype(jnp.float32) + 1.0
    b2 = ((hsh >> 23) % np.uint32(512)).astype(jnp.float32) + 1.0
    scale = validf / (b0 + b1 + b2)
    return tri, b0 * scale, b1 * scale, b2 * scale


def _body(table, out, idxA, idxB_, wA, wB, gA, gB, tA, tB, gsA, gsB, osA, osB):
    wid = lax.axis_index("s") * NC + lax.axis_index("c")
    pix0 = wid * PIX_PER_W
    b = pix0 // HWPIX
    y0 = (pix0 % HWPIX) // W
    lane = lax.broadcasted_iota(jnp.int32, (16,), 0)
    lane_u = lane.astype(jnp.uint32)

    def gen(c, idxbuf, wbuf):
        pbase = pix0 + c * C

        def one(g, _):
            pvec = (pbase + g * L).astype(jnp.uint32) + lane_u
            tri, w0, w1, w2 = _hash_pix(pvec)
            idxbuf[g // (IDXB // L), pl.ds((g % (IDXB // L)) * L, L)] = tri
            wbuf[0, pl.ds(g * L, L)] = w0
            wbuf[1, pl.ds(g * L, L)] = w1
            wbuf[2, pl.ds(g * L, L)] = w2
            return 0

        lax.fori_loop(0, NG, one, 0, unroll=4)

    def fire_gather(idxbuf, gbuf, gsem):
        for j in range(NIDX):
            pltpu.async_copy(
                table.at[idxbuf.at[j]], gbuf.at[pl.ds(j * IDXB, IDXB)], gsem
            )

    def wsum(idxbuf, gbuf, gsem, wbuf, tbuf):
        def one(g, _):
            rowv = g * L + lane
            w0 = wbuf[0, pl.ds(g * L, L)]
            w1 = wbuf[1, pl.ds(g * L, L)]
            w2 = wbuf[2, pl.ds(g * L, L)]
            accs = []
            for d in range(16):
                r0 = plsc.load_gather(gbuf, [rowv, lane * 0 + d])
                r1 = plsc.load_gather(gbuf, [rowv, lane * 0 + (16 + d)])
                r2 = plsc.load_gather(gbuf, [rowv, lane * 0 + (32 + d)])
                accs.append(w0 * r0 + w1 * r1 + w2 * r2)
            for d in range(16):
                tbuf[d, g // 8, pl.ds((g % 8) * L, L)] = accs[d]
            tbuf[16, g // 8, pl.ds((g % 8) * L, L)] = (w0 + w1) + w2
            return 0

        for j in range(NIDX):
            pltpu.make_async_copy(
                table.at[idxbuf.at[j]], gbuf.at[pl.ds(j * IDXB, IDXB)], gsem
            ).wait()
        lax.fori_loop(0, NG, one, 0)

    def out_slice(c):
        y = y0 + c
        return out.at[b, :, y // 8, :, y % 8, :]

    def fire_out(c, tbuf, osem):
        pltpu.async_copy(tbuf, out_slice(c), osem)

    def drain_out(tbuf, osem):
        pltpu.make_async_copy(tbuf, out_slice(0), osem).wait()

    gen(0, idxA, wA)
    fire_gather(idxA, gA, gsA)

    def step(t, _):
        c0 = 2 * t
        c1 = c0 + 1
        gen(c1, idxB_, wB)
        fire_gather(idxB_, gB, gsB)

        @pl.when(t > 0)
        def _():
            drain_out(tA, osA)

        wsum(idxA, gA, gsA, wA, tA)
        fire_out(c0, tA, osA)

        @pl.when(c0 + 2 < NCHUNK)
        def _():
            gen(c0 + 2, idxA, wA)
            fire_gather(idxA, gA, gsA)

        @pl.when(t > 0)
        def _():
            drain_out(tB, osB)

        wsum(idxB_, gB, gsB, wB, tB)
        fire_out(c1, tB, osB)
        return 0

    lax.fori_loop(0, NCHUNK // 2, step, 0)
    drain_out(tA, osA)
    drain_out(tB, osB)


@jax.jit
def _sc_render(table):
    mesh = plsc.VectorSubcoreMesh(core_axis_name="c", subcore_axis_name="s")
    return pl.kernel(
        _body,
        out_type=jax.ShapeDtypeStruct(
            (B, 17, H // 8, W // 128, 8, 128), jnp.float32
        ),
        mesh=mesh,
        scratch_types=[
            pltpu.VMEM((NIDX, IDXB), jnp.int32),
            pltpu.VMEM((NIDX, IDXB), jnp.int32),
            pltpu.VMEM((3, C), jnp.float32),
            pltpu.VMEM((3, C), jnp.float32),
            pltpu.VMEM((C, 48), jnp.float32),
            pltpu.VMEM((C, 48), jnp.float32),
            pltpu.VMEM((17, W // 128, 128), jnp.float32),
            pltpu.VMEM((17, W // 128, 128), jnp.float32),
            pltpu.SemaphoreType.DMA,
            pltpu.SemaphoreType.DMA,
            pltpu.SemaphoreType.DMA,
            pltpu.SemaphoreType.DMA,
        ],
        compiler_params=pltpu.CompilerParams(
            use_tc_tiling_on_sc=False, needs_layout_passes=False
        ),
    )(table)


def _detile_body(i6, o4):
    x = i6[0, 0]
    for xt in range(W // 128):
        o4[0, 0, :, pl.ds(xt * 128, 128)] = x[:, xt].reshape(H, 128)


@jax.jit
def _tc_detile(o6):
    return pl.pallas_call(
        _detile_body,
        grid=(B, 17),
        in_specs=[
            pl.BlockSpec(
                (1, 1, H // 8, W // 128, 8, 128), lambda b, d: (b, d, 0, 0, 0, 0)
            )
        ],
        out_specs=pl.BlockSpec((1, 1, H, W), lambda b, d: (b, d, 0, 0)),
        out_shape=jax.ShapeDtypeStruct((B, 17, H, W), jnp.float32),
    )(o6)


def kernel(v, f, attrs):
    del v, f
    table = attrs.reshape(NTAB, 48)
    o6 = _sc_render(table)
    return _tc_detile(o6)


# speedup vs baseline: 1.0622x; 1.0622x over previous
import numpy as np

import jax
import jax.numpy as jnp
from jax import lax
from jax.experimental import pallas as pl
from jax.experimental.pallas import tpu as pltpu
from jax.experimental.pallas import tpu_sc as plsc

H = 512
W = 512
B = 4
NF = 100000
NTAB = B * NF
HWPIX = H * W
NP = B * HWPIX

NC, NS, L = 2, 16, 16
NW = NC * NS
PIX_PER_W = NP // NW
C = 512
NG = C // L
IDXB = 128
NIDX = C // IDXB
NCHUNK = PIX_PER_W // C

_MUL = np.uint32(2654435761)


def _hash_pix(pvec_u32):
    hsh = pvec_u32 * _MUL
    tri = (hsh % np.uint32(NTAB)).astype(jnp.int32)
    valid = (hsh % np.uint32(7)) != np.uint32(0)
    validf = jnp.where(valid, np.float32(1.0), np.float32(0.0))
    b0 = ((hsh >> 3) % np.uint32(1024)).astype(jnp.float32) + 1.0
    b1 = ((hsh >> 13) % np.uint32(1024)).astype(jnp.float32) + 1.0
    b2 = ((hsh >> 23) % np.uint32(512)).astype(jnp.float32) + 1.0
    scale = validf / (b0 + b1 + b2)
    return tri, b0 * scale, b1 * scale, b2 * scale


def _body(
    table, out,
    idx0, idx1, idx2, idx3, w0_, w1_, w2_, w3_, g0, g1, g2, g3, t0, t1,
    gs0, gs1, gs2, gs3, os0, os1,
):
    idxs = [idx0, idx1, idx2, idx3]
    ws = [w0_, w1_, w2_, w3_]
    gs = [g0, g1, g2, g3]
    ts = [t0, t1]
    gsems = [gs0, gs1, gs2, gs3]
    osems = [os0, os1]
    wid = lax.axis_index("s") * NC + lax.axis_index("c")
    pix0 = wid * PIX_PER_W
    b = pix0 // HWPIX
    y0 = (pix0 % HWPIX) // W
    lane = lax.broadcasted_iota(jnp.int32, (16,), 0)
    lane_u = lane.astype(jnp.uint32)

    def gen(c, idxbuf, wbuf):
        pbase = pix0 + c * C

        def one(g, _):
            pvec = (pbase + g * L).astype(jnp.uint32) + lane_u
            tri, w0, w1, w2 = _hash_pix(pvec)
            idxbuf[g // (IDXB // L), pl.ds((g % (IDXB // L)) * L, L)] = tri
            wbuf[0, pl.ds(g * L, L)] = w0
            wbuf[1, pl.ds(g * L, L)] = w1
            wbuf[2, pl.ds(g * L, L)] = w2
            return 0

        lax.fori_loop(0, NG, one, 0, unroll=4)

    def fire_gather(idxbuf, gbuf, gsem):
        for j in range(NIDX):
            pltpu.async_copy(
                table.at[idxbuf.at[j]], gbuf.at[pl.ds(j * IDXB, IDXB)], gsem
            )

    def wsum(idxbuf, gbuf, gsem, wbuf, tbuf):
        def one(g, _):
            rowv = g * L + lane
            w0 = wbuf[0, pl.ds(g * L, L)]
            w1 = wbuf[1, pl.ds(g * L, L)]
            w2 = wbuf[2, pl.ds(g * L, L)]
            accs = []
            for d in range(16):
                r0 = plsc.load_gather(gbuf, [rowv, lane * 0 + d])
                r1 = plsc.load_gather(gbuf, [rowv, lane * 0 + (16 + d)])
                r2 = plsc.load_gather(gbuf, [rowv, lane * 0 + (32 + d)])
                accs.append(w0 * r0 + w1 * r1 + w2 * r2)
            for d in range(16):
                tbuf[d, g // 8, pl.ds((g % 8) * L, L)] = accs[d]
            tbuf[16, g // 8, pl.ds((g % 8) * L, L)] = (w0 + w1) + w2
            return 0

        for j in range(NIDX):
            pltpu.make_async_copy(
                table.at[idxbuf.at[j]], gbuf.at[pl.ds(j * IDXB, IDXB)], gsem
            ).wait()
        lax.fori_loop(0, NG, one, 0)

    def out_slice(c):
        y = y0 + c
        return out.at[b, :, y // 8, :, y % 8, :]

    def fire_out(c, tbuf, osem):
        pltpu.async_copy(tbuf, out_slice(c), osem)

    def drain_out(tbuf, osem):
        pltpu.make_async_copy(tbuf, out_slice(0), osem).wait()

    for c in range(3):
        gen(c, idxs[c], ws[c])
        fire_gather(idxs[c], gs[c], gsems[c])

    def step(t, _):
        for j in range(4):
            c = 4 * t + j
            jt = j % 2

            if j < 2:
                @pl.when(t > 0)
                def _():
                    drain_out(ts[jt], osems[jt])
            else:
                drain_out(ts[jt], osems[jt])

            wsum(idxs[j], gs[j], gsems[j], ws[j], ts[jt])
            fire_out(c, ts[jt], osems[jt])

            jn = (j + 3) % 4

            @pl.when(c + 3 < NCHUNK)
            def _():
                gen(c + 3, idxs[jn], ws[jn])
                fire_gather(idxs[jn], gs[jn], gsems[jn])

        return 0

    lax.fori_loop(0, NCHUNK // 4, step, 0)
    drain_out(t0, os0)
    drain_out(t1, os1)


@jax.jit
def _sc_render(table):
    mesh = plsc.VectorSubcoreMesh(core_axis_name="c", subcore_axis_name="s")
    return pl.kernel(
        _body,
        out_type=jax.ShapeDtypeStruct(
            (B, 17, H // 8, W // 128, 8, 128), jnp.float32
        ),
        mesh=mesh,
        scratch_types=(
            [pltpu.VMEM((NIDX, IDXB), jnp.int32)] * 4
            + [pltpu.VMEM((3, C), jnp.float32)] * 4
            + [pltpu.VMEM((C, 48), jnp.float32)] * 4
            + [pltpu.VMEM((17, W // 128, 128), jnp.float32)] * 2
            + [pltpu.SemaphoreType.DMA] * 6
        ),
        compiler_params=pltpu.CompilerParams(
            use_tc_tiling_on_sc=False, needs_layout_passes=False
        ),
    )(table)


def kernel(v, f, attrs):
    del v, f
    table = attrs.reshape(NTAB, 48)
    o6 = _sc_render(table)
    return o6.transpose(0, 1, 2, 4, 3, 5).reshape(B, 17, H, W)


# speedup vs baseline: 1.0856x; 1.0221x over previous
import numpy as np

import jax
import jax.numpy as jnp
from jax import lax
from jax.experimental import pallas as pl
from jax.experimental.pallas import tpu as pltpu
from jax.experimental.pallas import tpu_sc as plsc

H = 512
W = 512
B = 4
NF = 100000
NTAB = B * NF
HWPIX = H * W
NP = B * HWPIX

NC, NS, L = 2, 16, 16
NW = NC * NS
PIX_PER_W = NP // NW
C = 512
NG = C // L
IDXB = 128
NIDX = C // IDXB
NCHUNK = PIX_PER_W // C

_MUL = np.uint32(2654435761)


def _hash_pix(pvec_u32):
    hsh = pvec_u32 * _MUL
    tri = (hsh % np.uint32(NTAB)).astype(jnp.int32)
    valid = (hsh % np.uint32(7)) != np.uint32(0)
    validf = jnp.where(valid, np.float32(1.0), np.float32(0.0))
    b0 = ((hsh >> 3) % np.uint32(1024)).astype(jnp.float32) + 1.0
    b1 = ((hsh >> 13) % np.uint32(1024)).astype(jnp.float32) + 1.0
    b2 = ((hsh >> 23) % np.uint32(512)).astype(jnp.float32) + 1.0
    scale = validf / (b0 + b1 + b2)
    return tri, b0 * scale, b1 * scale, b2 * scale


def _body(table, out, idxA, idxB_, wA, wB, gA, gB, tA, tB, gsA, gsB, osA, osB):
    wid = lax.axis_index("s") * NC + lax.axis_index("c")
    pix0 = wid * PIX_PER_W
    b = pix0 // HWPIX
    y0 = (pix0 % HWPIX) // W
    lane = lax.broadcasted_iota(jnp.int32, (16,), 0)
    lane_u = lane.astype(jnp.uint32)

    def gen(c, idxbuf, wbuf):
        pbase = pix0 + c * C

        def one(g, _):
            pvec = (pbase + g * L).astype(jnp.uint32) + lane_u
            tri, w0, w1, w2 = _hash_pix(pvec)
            idxbuf[g // (IDXB // L), pl.ds((g % (IDXB // L)) * L, L)] = tri
            wbuf[0, pl.ds(g * L, L)] = w0
            wbuf[1, pl.ds(g * L, L)] = w1
            wbuf[2, pl.ds(g * L, L)] = w2
            return 0

        lax.fori_loop(0, NG, one, 0, unroll=4)

    def fire_gather(idxbuf, gbuf, gsem):
        for j in range(NIDX):
            pltpu.async_copy(
                table.at[idxbuf.at[j]], gbuf.at[pl.ds(j * IDXB, IDXB)], gsem
            )

    def wsum(idxbuf, gbuf, gsem, wbuf, tbuf):
        def one(g, _):
            rowv = g * L + lane
            w0 = wbuf[0, pl.ds(g * L, L)]
            w1 = wbuf[1, pl.ds(g * L, L)]
            w2 = wbuf[2, pl.ds(g * L, L)]
            accs = []
            for d in range(16):
                r0 = plsc.load_gather(gbuf, [rowv, lane * 0 + d])
                r1 = plsc.load_gather(gbuf, [rowv, lane * 0 + (16 + d)])
                r2 = plsc.load_gather(gbuf, [rowv, lane * 0 + (32 + d)])
                accs.append(w0 * r0 + w1 * r1 + w2 * r2)
            for d in range(16):
                tbuf[d, g // 8, pl.ds((g % 8) * L, L)] = accs[d]
            tbuf[16, g // 8, pl.ds((g % 8) * L, L)] = (w0 + w1) + w2
            return 0

        for j in range(NIDX):
            pltpu.make_async_copy(
                table.at[idxbuf.at[j]], gbuf.at[pl.ds(j * IDXB, IDXB)], gsem
            ).wait()
        lax.fori_loop(0, NG, one, 0)

    def out_slice(c):
        y = y0 + c
        return out.at[b, :, y // 8, :, y % 8, :]

    def fire_out(c, tbuf, osem):
        pltpu.async_copy(tbuf, out_slice(c), osem)

    def drain_out(tbuf, osem):
        pltpu.make_async_copy(tbuf, out_slice(0), osem).wait()

    gen(0, idxA, wA)
    fire_gather(idxA, gA, gsA)

    def step(t, _):
        c0 = 2 * t
        c1 = c0 + 1
        gen(c1, idxB_, wB)
        fire_gather(idxB_, gB, gsB)

        @pl.when(t > 0)
        def _():
            drain_out(tA, osA)

        wsum(idxA, gA, gsA, wA, tA)
        fire_out(c0, tA, osA)

        @pl.when(c0 + 2 < NCHUNK)
        def _():
            gen(c0 + 2, idxA, wA)
            fire_gather(idxA, gA, gsA)

        @pl.when(t > 0)
        def _():
            drain_out(tB, osB)

        wsum(idxB_, gB, gsB, wB, tB)
        fire_out(c1, tB, osB)
        return 0

    lax.fori_loop(0, NCHUNK // 2, step, 0)
    drain_out(tA, osA)
    drain_out(tB, osB)


@jax.jit
def _sc_render(table):
    mesh = plsc.VectorSubcoreMesh(core_axis_name="c", subcore_axis_name="s")
    return pl.kernel(
        _body,
        out_type=jax.ShapeDtypeStruct(
            (B, 17, H // 8, W // 128, 8, 128), jnp.float32
        ),
        mesh=mesh,
        scratch_types=(
            [pltpu.VMEM((NIDX, IDXB), jnp.int32)] * 2
            + [pltpu.VMEM((3, C), jnp.float32)] * 2
            + [pltpu.VMEM((C, 48), jnp.float32)] * 2
            + [pltpu.VMEM((17, W // 128, 128), jnp.float32)] * 2
            + [pltpu.SemaphoreType.DMA] * 4
        ),
        compiler_params=pltpu.CompilerParams(
            use_tc_tiling_on_sc=False, needs_layout_passes=False
        ),
    )(table)


def kernel(v, f, attrs):
    del v, f
    table = attrs.reshape(NTAB, 48)
    o6 = _sc_render(table)
    return o6.transpose(0, 1, 2, 4, 3, 5).reshape(B, 17, H, W)
